# Initial kernel scaffold; baseline (speedup 1.0000x reference)
import jax
import jax.numpy as jnp
from jax.experimental import pallas as pl


def kernel(R):
    raise NotImplementedError("write your pallas kernel here")


# speedup vs baseline: 1.0068x; 1.0068x over previous
import jax
import jax.numpy as jnp
from jax.experimental import pallas as pl
from jax.experimental.pallas import tpu as pltpu

_L, _K, _D = 32, 1024, 2048
_LAMBDA = 0.1
_CC_LOW = 768
_CC_HIGH = 1024
_TOTAL_UNITS = 16 * _CC_LOW + 16 * _CC_HIGH
_ROWS = _L * _K
_BK = 512
_NBLK = _ROWS // _BK


def _block_kernel(x_ref, out_ref):
    i = pl.program_id(0)

    @pl.when(i == 0)
    def _init():
        out_ref[0, 0] = jnp.float32(0.0)

    x = x_ref[...]
    row_sum = jnp.sum(x, axis=1)

    abs_row = i * _BK + jax.lax.broadcasted_iota(jnp.int32, (_BK,), 0)
    k = jnp.bitwise_and(abs_row, _K - 1)
    col = jax.lax.broadcasted_iota(jnp.int32, (_BK, _D), 1)
    diag_mask = col == k[:, None]
    r_dk = jnp.sum(jnp.where(diag_mask, x, 0.0), axis=1)

    r_minus = (row_sum - r_dk) * jnp.float32(1.0 / (_D - 1))
    a = jnp.abs(r_dk)
    b = jnp.abs(r_minus)
    s = (a - b) / (a + b + jnp.float32(1e-6))

    cc = jnp.where(abs_row < 16 * _K, _CC_LOW, _CC_HIGH)
    valid = k < cc
    partial = jnp.sum(jnp.where(valid, s, 0.0))
    out_ref[0, 0] += partial


def kernel(R):
    flat = R.reshape(_ROWS, _D)
    total = pl.pallas_call(
        _block_kernel,
        grid=(_NBLK,),
        in_specs=[pl.BlockSpec((_BK, _D), lambda i: (i, 0))],
        out_specs=pl.BlockSpec(
            (1, 1), lambda i: (0, 0), memory_space=pltpu.SMEM
        ),
        out_shape=jax.ShapeDtypeStruct((1, 1), jnp.float32),
    )(flat)
    return total[0, 0] * jnp.float32(-_LAMBDA / _TOTAL_UNITS)
